# Initial kernel scaffold; baseline (speedup 1.0000x reference)
import jax
import jax.numpy as jnp
from jax.experimental import pallas as pl


def kernel(x, edge_index, W_self1, W_neigh1, b1, W_self2, W_neigh2, b2, W_self3, W_neigh3, b3):
    raise NotImplementedError("write your pallas kernel here")


# speedup vs baseline: 10.1050x; 10.1050x over previous
import functools

import jax
import jax.numpy as jnp
from jax import lax
from jax.experimental import pallas as pl
from jax.experimental.pallas import tpu as pltpu
from jax.experimental.pallas import tpu_sc as plsc

N = 10000
E = 320000
IN = 128
HID = 32
OUT = 128

NC, NS = 2, 16
NW = NC * NS
CHUNK = 128
NCH = 79
EP = NW * NCH * CHUNK
PAD = EP - E
ACC_N = 10112
RPT = 632

_MESH = plsc.VectorSubcoreMesh(core_axis_name="c", subcore_axis_name="s",
                               num_cores=NC, num_subcores=NS)
_SC_PARAMS = pltpu.CompilerParams(use_tc_tiling_on_sc=False)


def _sc_body(with_deg, feat_hbm, src_hbm, dst_hbm, zacc_hbm, zdeg_hbm, ones_hbm,
             acc_out, deg_out, src_v, dst_v, rows_v, ones_v,
             acc_sh, deg_sh, sem):
    c = lax.axis_index("c")
    s = lax.axis_index("s")
    w = c * NS + s

    zrows = ACC_N // NS
    pltpu.sync_copy(zacc_hbm.at[pl.ds(s * zrows, zrows)],
                    acc_sh.at[pl.ds(s * zrows, zrows)])
    if with_deg:
        pltpu.sync_copy(zdeg_hbm.at[pl.ds(s * zrows, zrows)],
                        deg_sh.at[pl.ds(s * zrows, zrows)])
        pltpu.sync_copy(ones_hbm, ones_v)
    pltpu.sync_copy(src_hbm.at[w], src_v)
    pltpu.sync_copy(dst_hbm.at[w], dst_v)
    plsc.subcore_barrier()

    def chunk_body(j, carry):
        pltpu.async_copy(feat_hbm.at[src_v.at[j, 0]], rows_v, sem).wait()
        pltpu.sync_copy(rows_v, acc_sh.at[dst_v.at[j, 0]], add=True)
        if with_deg:
            pltpu.sync_copy(ones_v, deg_sh.at[dst_v.at[j, 0]], add=True)
        return carry

    lax.fori_loop(0, NCH, chunk_body, 0)
    plsc.subcore_barrier()

    @pl.when(s < NS - 1)
    def _():
        pltpu.sync_copy(acc_sh.at[pl.ds(s * RPT, RPT)],
                        acc_out.at[c, pl.ds(s * RPT, RPT)])
        if with_deg:
            pltpu.sync_copy(deg_sh.at[pl.ds(s * RPT, RPT)],
                            deg_out.at[c, pl.ds(s * RPT, RPT)])

    @pl.when(s == NS - 1)
    def _():
        last = N - (NS - 1) * RPT
        pltpu.sync_copy(acc_sh.at[pl.ds((NS - 1) * RPT, last)],
                        acc_out.at[c, pl.ds((NS - 1) * RPT, last)])
        if with_deg:
            pltpu.sync_copy(deg_sh.at[pl.ds((NS - 1) * RPT, last)],
                            deg_out.at[c, pl.ds((NS - 1) * RPT, last)])


_SC_SCRATCH = [
    pltpu.VMEM((NCH, 1, CHUNK), jnp.int32),
    pltpu.VMEM((NCH, 1, CHUNK), jnp.int32),
    pltpu.VMEM((CHUNK, HID), jnp.float32),
    pltpu.VMEM((CHUNK, 16), jnp.float32),
    pltpu.VMEM_SHARED((ACC_N, HID), jnp.float32),
    pltpu.VMEM_SHARED((ACC_N, 16), jnp.float32),
    pltpu.SemaphoreType.DMA,
]

_sc_agg_deg = functools.partial(
    pl.kernel,
    out_type=(jax.ShapeDtypeStruct((NC, N, HID), jnp.float32),
              jax.ShapeDtypeStruct((NC, N, 16), jnp.float32)),
    mesh=_MESH,
    scratch_types=_SC_SCRATCH,
    compiler_params=_SC_PARAMS,
)(functools.partial(_sc_body, True))


def _sc_body_nodeg(feat_hbm, src_hbm, dst_hbm, zacc_hbm, acc_out,
                   src_v, dst_v, rows_v, acc_sh, sem):
    _sc_body(False, feat_hbm, src_hbm, dst_hbm, zacc_hbm, None, None,
             acc_out, None, src_v, dst_v, rows_v, None,
             acc_sh, None, sem)


_sc_agg = functools.partial(
    pl.kernel,
    out_type=jax.ShapeDtypeStruct((NC, N, HID), jnp.float32),
    mesh=_MESH,
    scratch_types=[
        pltpu.VMEM((NCH, 1, CHUNK), jnp.int32),
        pltpu.VMEM((NCH, 1, CHUNK), jnp.int32),
        pltpu.VMEM((CHUNK, HID), jnp.float32),
        pltpu.VMEM_SHARED((ACC_N, HID), jnp.float32),
        pltpu.SemaphoreType.DMA,
    ],
    compiler_params=_SC_PARAMS,
)(_sc_body_nodeg)


RB = 1000
GRID = N // RB


def _mm2_body(x_ref, wa_ref, wb_ref, oa_ref, ob_ref):
    x = x_ref[...]
    oa_ref[...] = jnp.dot(x, wa_ref[...], preferred_element_type=jnp.float32)
    ob_ref[...] = jnp.dot(x, wb_ref[...], preferred_element_type=jnp.float32)


def _tc_mm2(x, wa, wb):
    return pl.pallas_call(
        _mm2_body,
        grid=(GRID,),
        in_specs=[
            pl.BlockSpec((RB, IN), lambda i: (i, 0)),
            pl.BlockSpec((IN, HID), lambda i: (0, 0)),
            pl.BlockSpec((IN, HID), lambda i: (0, 0)),
        ],
        out_specs=[
            pl.BlockSpec((RB, HID), lambda i: (i, 0)),
            pl.BlockSpec((RB, HID), lambda i: (i, 0)),
        ],
        out_shape=[jax.ShapeDtypeStruct((N, HID), jnp.float32),
                   jax.ShapeDtypeStruct((N, HID), jnp.float32)],
    )(x, wa, wb)


def _combine(acc_ref, deg_ref):
    agg = acc_ref[0] + acc_ref[1]
    deg = deg_ref[0, :, 0:1] + deg_ref[1, :, 0:1]
    return agg / jnp.maximum(deg, 1.0)


def _layer_mid_body(s_ref, acc_ref, deg_ref, b_ref, ws_ref, wn_ref,
                    os_ref, op_ref):
    h = jnp.maximum(s_ref[...] + _combine(acc_ref, deg_ref) + b_ref[...], 0.0)
    os_ref[...] = jnp.dot(h, ws_ref[...], preferred_element_type=jnp.float32)
    op_ref[...] = jnp.dot(h, wn_ref[...], preferred_element_type=jnp.float32)


def _tc_layer_mid(sprev, acc, deg, b, ws, wn, ws_out_dim):
    return pl.pallas_call(
        _layer_mid_body,
        grid=(GRID,),
        in_specs=[
            pl.BlockSpec((RB, HID), lambda i: (i, 0)),
            pl.BlockSpec((NC, RB, HID), lambda i: (0, i, 0)),
            pl.BlockSpec((NC, RB, 16), lambda i: (0, i, 0)),
            pl.BlockSpec((1, HID), lambda i: (0, 0)),
            pl.BlockSpec((HID, ws_out_dim), lambda i: (0, 0)),
            pl.BlockSpec((HID, HID), lambda i: (0, 0)),
        ],
        out_specs=[
            pl.BlockSpec((RB, ws_out_dim), lambda i: (i, 0)),
            pl.BlockSpec((RB, HID), lambda i: (i, 0)),
        ],
        out_shape=[jax.ShapeDtypeStruct((N, ws_out_dim), jnp.float32),
                   jax.ShapeDtypeStruct((N, HID), jnp.float32)],
    )(sprev, acc, deg, b, ws, wn)


def _layer2_body(s_ref, acc_ref, deg_ref, b_ref, ws_ref, os_ref, oh_ref):
    h = jnp.maximum(s_ref[...] + _combine(acc_ref, deg_ref) + b_ref[...], 0.0)
    oh_ref[...] = h
    os_ref[...] = jnp.dot(h, ws_ref[...], preferred_element_type=jnp.float32)


def _tc_layer2(sprev, acc, deg, b, ws):
    return pl.pallas_call(
        _layer2_body,
        grid=(GRID,),
        in_specs=[
            pl.BlockSpec((RB, HID), lambda i: (i, 0)),
            pl.BlockSpec((NC, RB, HID), lambda i: (0, i, 0)),
            pl.BlockSpec((NC, RB, 16), lambda i: (0, i, 0)),
            pl.BlockSpec((1, HID), lambda i: (0, 0)),
            pl.BlockSpec((HID, OUT), lambda i: (0, 0)),
        ],
        out_specs=[
            pl.BlockSpec((RB, OUT), lambda i: (i, 0)),
            pl.BlockSpec((RB, HID), lambda i: (i, 0)),
        ],
        out_shape=[jax.ShapeDtypeStruct((N, OUT), jnp.float32),
                   jax.ShapeDtypeStruct((N, HID), jnp.float32)],
    )(sprev, acc, deg, b, ws)


def _layer3_body(s_ref, acc_ref, deg_ref, b_ref, wn_ref, o_ref):
    hn = _combine(acc_ref, deg_ref)
    o_ref[...] = (s_ref[...] + b_ref[...]
                  + jnp.dot(hn, wn_ref[...], preferred_element_type=jnp.float32))


def _tc_layer3(sprev, acc, deg, b, wn):
    return pl.pallas_call(
        _layer3_body,
        grid=(GRID,),
        in_specs=[
            pl.BlockSpec((RB, OUT), lambda i: (i, 0)),
            pl.BlockSpec((NC, RB, HID), lambda i: (0, i, 0)),
            pl.BlockSpec((NC, RB, 16), lambda i: (0, i, 0)),
            pl.BlockSpec((1, OUT), lambda i: (0, 0)),
            pl.BlockSpec((HID, OUT), lambda i: (0, 0)),
        ],
        out_specs=pl.BlockSpec((RB, OUT), lambda i: (i, 0)),
        out_shape=jax.ShapeDtypeStruct((N, OUT), jnp.float32),
    )(sprev, acc, deg, b, wn)


def kernel(x, edge_index, W_self1, W_neigh1, b1, W_self2, W_neigh2, b2,
           W_self3, W_neigh3, b3):
    src = jnp.concatenate(
        [edge_index[0].astype(jnp.int32), jnp.zeros((PAD,), jnp.int32)]
    ).reshape(NW, NCH, 1, CHUNK)
    dst = jnp.concatenate(
        [edge_index[1].astype(jnp.int32), jnp.full((PAD,), N, jnp.int32)]
    ).reshape(NW, NCH, 1, CHUNK)
    zacc = jnp.zeros((ACC_N, HID), jnp.float32)
    zdeg = jnp.zeros((ACC_N, 16), jnp.float32)
    ones = jnp.ones((CHUNK, 16), jnp.float32)

    s1, p1 = _tc_mm2(x, W_self1, W_neigh1)
    acc1, deg = _sc_agg_deg(p1, src, dst, zacc, zdeg, ones)
    s2, p2 = _tc_layer_mid(s1, acc1, deg, b1.reshape(1, HID),
                           W_self2, W_neigh2, HID)
    acc2 = _sc_agg(p2, src, dst, zacc)
    s3, h2 = _tc_layer2(s2, acc2, deg, b2.reshape(1, HID), W_self3)
    acc3 = _sc_agg(h2, src, dst, zacc)
    return _tc_layer3(s3, acc3, deg, b3.reshape(1, OUT), W_neigh3)
